# baseline (speedup 1.0000x reference)
import jax
import jax.numpy as jnp
from jax import lax
from jax.experimental import pallas as pl
from jax.experimental.pallas import tpu as pltpu
from jax.experimental.pallas import tpu_sc as plsc

BATCH = 16
N_IN = 480000
N_OUT = 160000
HALF_OUT = N_OUT // 2
CHUNK_OUT = 40000
NUM_CHUNKS = HALF_OUT // CHUNK_OUT
LANES = 16


def _sc_kernel(wav_hbm, out_hbm, idx_v, out_v, sem):
    nc = plsc.get_sparse_core_info().num_cores
    wid = lax.axis_index("s") * nc + lax.axis_index("c")
    row = wid // 2
    half = wid % 2
    out_base = row * N_OUT + half * HALF_OUT
    in_base = row * N_IN + half * 3 * HALF_OUT

    lane3 = 3 * lax.iota(jnp.int32, LANES)

    for c in range(NUM_CHUNKS):
        base = in_base + 3 * c * CHUNK_OUT

        @plsc.parallel_loop(0, CHUNK_OUT, step=LANES, unroll=8)
        def _(i):
            idx_v[pl.ds(i, LANES)] = base + 3 * i + lane3

        pltpu.async_copy(wav_hbm.at[idx_v], out_v, sem).wait()
        pltpu.sync_copy(
            out_v, out_hbm.at[pl.ds(out_base + c * CHUNK_OUT, CHUNK_OUT)])


@jax.jit
def _resample(wav_flat):
    mesh = plsc.VectorSubcoreMesh(core_axis_name="c", subcore_axis_name="s")
    return pl.kernel(
        _sc_kernel,
        mesh=mesh,
        out_type=jax.ShapeDtypeStruct((BATCH * N_OUT,), jnp.float32),
        scratch_types=[
            pltpu.VMEM((CHUNK_OUT,), jnp.int32),
            pltpu.VMEM((CHUNK_OUT,), jnp.float32),
            pltpu.SemaphoreType.DMA,
        ],
        compiler_params=pltpu.CompilerParams(needs_layout_passes=False),
    )(wav_flat)


def kernel(wav):
    wav = wav.reshape(wav.shape[0], -1)
    out = _resample(wav.reshape(-1))
    return out.reshape(wav.shape[0], N_OUT)


# speedup vs baseline: 4.3313x; 4.3313x over previous
import jax
import jax.numpy as jnp
from jax import lax
from jax.experimental import pallas as pl
from jax.experimental.pallas import tpu as pltpu
from jax.experimental.pallas import tpu_sc as plsc

BATCH = 16
N_IN = 480000
N_OUT = 160000
HALF_OUT = N_OUT // 2
CHUNK_OUT = 32000
CHUNK_IN = 3 * CHUNK_OUT
CHUNKS = (32000, 32000, 16000)
LANES = 16


def _sc_kernel(wav_hbm, out_hbm, in_v, out_v):
    nc = plsc.get_sparse_core_info().num_cores
    wid = lax.axis_index("s") * nc + lax.axis_index("c")
    row = wid // 2
    half = wid % 2
    out_base = half * HALF_OUT

    lane3 = 3 * lax.iota(jnp.int32, LANES)

    for c, width in enumerate(CHUNKS):
        out_off = out_base + sum(CHUNKS[:c])
        in_off = 3 * out_off
        pltpu.sync_copy(wav_hbm.at[row, pl.ds(in_off, 3 * width)],
                        in_v.at[pl.ds(0, 3 * width)])

        @plsc.parallel_loop(0, width, step=LANES, unroll=8)
        def _(i):
            out_v[pl.ds(i, LANES)] = plsc.load_gather(in_v, [lane3 + 3 * i])

        pltpu.sync_copy(out_v.at[pl.ds(0, width)],
                        out_hbm.at[row, pl.ds(out_off, width)])


@jax.jit
def _resample(wav):
    mesh = plsc.VectorSubcoreMesh(core_axis_name="c", subcore_axis_name="s")
    return pl.kernel(
        _sc_kernel,
        mesh=mesh,
        out_type=jax.ShapeDtypeStruct((BATCH, N_OUT), jnp.float32),
        scratch_types=[
            pltpu.VMEM((CHUNK_IN,), jnp.float32),
            pltpu.VMEM((CHUNK_OUT,), jnp.float32),
        ],
        compiler_params=pltpu.CompilerParams(needs_layout_passes=False),
    )(wav)


def kernel(wav):
    wav = wav.reshape(wav.shape[0], -1)
    return _resample(wav)
